# Initial kernel scaffold; baseline (speedup 1.0000x reference)
import jax
import jax.numpy as jnp
from jax.experimental import pallas as pl


def kernel(x, Wr1, br1, Wg, We_in, We_out):
    raise NotImplementedError("write your pallas kernel here")


# speedup vs baseline: 1.7978x; 1.7978x over previous
import functools

import jax
import jax.numpy as jnp
from jax.experimental import pallas as pl
from jax.experimental.pallas import tpu as pltpu

E = 8
K = 2
D = 768
H = 768
TS = 256


def _moe_block_kernel(x_ref, wr1_ref, br1_ref, wg_ref, win_ref, wout_ref, y_ref):
    xb = x_ref[...]
    h = jax.lax.dot_general(
        xb, wr1_ref[...], (((1,), (1,)), ((), ())),
        preferred_element_type=jnp.float32,
    )
    h = jnp.tanh(h + br1_ref[...])
    logits = jax.lax.dot_general(
        h, wg_ref[...], (((1,), (1,)), ((), ())),
        preferred_element_type=jnp.float32,
    )
    m = jnp.max(logits, axis=1, keepdims=True)
    p = jnp.exp(logits - m)
    p = p / jnp.sum(p, axis=1, keepdims=True)
    e_iota = jax.lax.broadcasted_iota(jnp.int32, p.shape, 1)
    m1 = jnp.max(p, axis=1, keepdims=True)
    i1 = jnp.min(jnp.where(p == m1, e_iota, E), axis=1, keepdims=True)
    p_rest = jnp.where(e_iota == i1, -jnp.inf, p)
    m2 = jnp.max(p_rest, axis=1, keepdims=True)
    i2 = jnp.min(jnp.where(p_rest == m2, e_iota, E), axis=1, keepdims=True)
    denom = m1 + m2 + 1e-6
    gates = jnp.where(e_iota == i1, m1 / denom,
                      jnp.where(e_iota == i2, m2 / denom, 0.0))

    xbb = xb.astype(jnp.bfloat16)
    acc = jnp.zeros((TS, D), jnp.float32)
    for e in range(E):
        hh = jax.lax.dot(xbb, win_ref[e], preferred_element_type=jnp.float32)
        hh = jnp.maximum(hh, 0.0).astype(jnp.bfloat16)
        oo = jax.lax.dot(hh, wout_ref[e], preferred_element_type=jnp.float32)
        acc = acc + gates[:, e:e + 1] * oo
    y_ref[...] = acc


@functools.partial(jax.jit, static_argnames=("interpret",))
def _moe(x2d, Wr1, br1_2d, Wg, win_bf16, wout_bf16, interpret=False):
    S = x2d.shape[0]
    grid = (S // TS,)
    y = pl.pallas_call(
        _moe_block_kernel,
        grid=grid,
        in_specs=[
            pl.BlockSpec((TS, D), lambda i: (i, 0)),
            pl.BlockSpec((D, D), lambda i: (0, 0)),
            pl.BlockSpec((1, D), lambda i: (0, 0)),
            pl.BlockSpec((E, D), lambda i: (0, 0)),
            pl.BlockSpec((E, D, H), lambda i: (0, 0, 0)),
            pl.BlockSpec((E, H, D), lambda i: (0, 0, 0)),
        ],
        out_specs=pl.BlockSpec((TS, D), lambda i: (i, 0)),
        out_shape=jax.ShapeDtypeStruct((S, D), jnp.float32),
        compiler_params=pltpu.CompilerParams(
            dimension_semantics=("arbitrary",),
            vmem_limit_bytes=60 * 1024 * 1024,
        ),
        interpret=interpret,
    )(x2d, Wr1, br1_2d, Wg, win_bf16, wout_bf16)
    return y


def kernel(x, Wr1, br1, Wg, We_in, We_out, interpret=False):
    bsz, length, d = x.shape
    x2d = x.reshape(-1, d)
    y = _moe(x2d, Wr1, br1.reshape(1, -1), Wg,
             We_in.astype(jnp.bfloat16), We_out.astype(jnp.bfloat16),
             interpret=interpret)
    loss = jnp.zeros((), dtype=jnp.float32)
    return y.reshape(bsz, length, d), loss


# speedup vs baseline: 2.1393x; 1.1899x over previous
import functools

import jax
import jax.numpy as jnp
from jax.experimental import pallas as pl
from jax.experimental.pallas import tpu as pltpu

E = 8
K = 2
D = 768
H = 768
TS = 256


def _moe_block_kernel(x_ref, wr1_ref, br1_ref, wg_ref, win_ref, wout_ref, y_ref):
    xb = x_ref[...]
    h = jax.lax.dot_general(
        xb, wr1_ref[...], (((1,), (1,)), ((), ())),
        preferred_element_type=jnp.float32,
    )
    h = jnp.tanh(h + br1_ref[...])
    logits = jax.lax.dot_general(
        h, wg_ref[...], (((1,), (1,)), ((), ())),
        preferred_element_type=jnp.float32,
    )
    m = jnp.max(logits, axis=1, keepdims=True)
    p = jnp.exp(logits - m)
    p = p / jnp.sum(p, axis=1, keepdims=True)
    e_iota = jax.lax.broadcasted_iota(jnp.int32, p.shape, 1)
    m1 = jnp.max(p, axis=1, keepdims=True)
    i1 = jnp.min(jnp.where(p == m1, e_iota, E), axis=1, keepdims=True)
    p_rest = jnp.where(e_iota == i1, -jnp.inf, p)
    m2 = jnp.max(p_rest, axis=1, keepdims=True)
    i2 = jnp.min(jnp.where(p_rest == m2, e_iota, E), axis=1, keepdims=True)
    denom = m1 + m2 + 1e-6
    gates = jnp.where(e_iota == i1, m1 / denom,
                      jnp.where(e_iota == i2, m2 / denom, 0.0))

    acc = jnp.zeros((TS, D), jnp.float32)
    for e in range(E):
        hh = jax.lax.dot(xb, win_ref[e], preferred_element_type=jnp.float32)
        hh = jnp.maximum(hh, 0.0)
        oo = jax.lax.dot(hh, wout_ref[e], preferred_element_type=jnp.float32)
        acc = acc + gates[:, e:e + 1] * oo
    y_ref[...] = acc


@functools.partial(jax.jit, static_argnames=("interpret",))
def _moe(x2d, Wr1, br1_2d, Wg, win_bf16, wout_bf16, interpret=False):
    S = x2d.shape[0]
    grid = (S // TS,)
    y = pl.pallas_call(
        _moe_block_kernel,
        grid=grid,
        in_specs=[
            pl.BlockSpec((TS, D), lambda i: (i, 0)),
            pl.BlockSpec((D, D), lambda i: (0, 0)),
            pl.BlockSpec((1, D), lambda i: (0, 0)),
            pl.BlockSpec((E, D), lambda i: (0, 0)),
            pl.BlockSpec((E, D, H), lambda i: (0, 0, 0)),
            pl.BlockSpec((E, H, D), lambda i: (0, 0, 0)),
        ],
        out_specs=pl.BlockSpec((TS, D), lambda i: (i, 0)),
        out_shape=jax.ShapeDtypeStruct((S, D), jnp.float32),
        compiler_params=pltpu.CompilerParams(
            dimension_semantics=("arbitrary",),
            vmem_limit_bytes=60 * 1024 * 1024,
        ),
        interpret=interpret,
    )(x2d, Wr1, br1_2d, Wg, win_bf16, wout_bf16)
    return y


def kernel(x, Wr1, br1, Wg, We_in, We_out, interpret=False):
    bsz, length, d = x.shape
    x2d = x.reshape(-1, d)
    y = _moe(x2d, Wr1, br1.reshape(1, -1), Wg, We_in, We_out,
             interpret=interpret)
    loss = jnp.zeros((), dtype=jnp.float32)
    return y.reshape(bsz, length, d), loss
